# baseline (speedup 1.0000x reference)
import jax
import jax.numpy as jnp
from jax import lax
from jax.experimental import pallas as pl
from jax.experimental.pallas import tpu as pltpu
from jax.experimental.pallas import tpu_sc as plsc

BL = 2048

NC, NS = 2, 16
NW = NC * NS
CHUNK = 8
SC_ROWS = 2048


def _add_kernel(x_ref, pe_ref, o_ref):
    o_ref[...] = x_ref[...] + pe_ref[...]


def _tc_partial(x, pe_table, L1):
    B, L, D = x.shape
    grid = (L1 // BL, B)
    return pl.pallas_call(
        _add_kernel,
        grid=grid,
        in_specs=[
            pl.BlockSpec((1, BL, D), lambda i, b: (b, i, 0)),
            pl.BlockSpec((BL, D), lambda i, b: (i, 0)),
        ],
        out_specs=pl.BlockSpec((1, BL, D), lambda i, b: (b, i, 0)),
        out_shape=jax.ShapeDtypeStruct((B, L, D), x.dtype),
    )(x, pe_table)


NBUF = 4


def _make_sc(B, L, D, L1):
    sc_rows = L - L1
    rows_per_w = sc_rows // NW
    n_chunks = rows_per_w // CHUNK
    T = n_chunks * B
    n_unroll = CHUNK * D // 16 // 8
    mesh = plsc.VectorSubcoreMesh(core_axis_name="c", subcore_axis_name="s")

    def body(x_hbm, pe_hbm, o_hbm, *scratch):
        in_bufs = scratch[0:NBUF]
        out_bufs = scratch[NBUF:2 * NBUF]
        pe_bufs = scratch[2 * NBUF:2 * NBUF + 2]
        in_sems = scratch[2 * NBUF + 2:3 * NBUF + 2]
        out_sems = scratch[3 * NBUF + 2:4 * NBUF + 2]
        pe_sems = scratch[4 * NBUF + 2:4 * NBUF + 4]

        w = lax.axis_index("s") * NC + lax.axis_index("c")
        l_base = L1 + w * rows_per_w

        def x_off(t):
            j, b = divmod(t, B)
            return (b * L + l_base + j * CHUNK) * D

        def o_off(t):
            j, b = divmod(t, B)
            return (b * sc_rows + (l_base - L1) + j * CHUNK) * D

        def pe_load(j):
            return pltpu.async_copy(
                pe_hbm.at[pl.ds((l_base + j * CHUNK) * D, CHUNK * D)],
                pe_bufs[j % 2],
                pe_sems[j % 2],
            )

        pe_d = {0: pe_load(0)}
        if n_chunks > 1:
            pe_d[1] = pe_load(1)
        in_d = {}
        for t in range(min(NBUF, T)):
            in_d[t] = pltpu.async_copy(
                x_hbm.at[pl.ds(x_off(t), CHUNK * D)], in_bufs[t % NBUF],
                in_sems[t % NBUF],
            )

        out_d = {}
        for t in range(T):
            j, b = divmod(t, B)
            slot = t % NBUF
            if b == 0:
                pe_d[j].wait()
            in_d[t].wait()
            if t >= NBUF:
                out_d[t - NBUF].wait()

            ib, ob, pb = in_bufs[slot], out_bufs[slot], pe_bufs[j % 2]

            def add_body(i, c, ib=ib, ob=ob, pb=pb):
                base = i * 128
                for k in range(8):
                    s = pl.ds(base + k * 16, 16)
                    ob[s] = ib[s] + pb[s]
                return c

            lax.fori_loop(0, n_unroll, add_body, 0)

            out_d[t] = pltpu.async_copy(
                ob, o_hbm.at[pl.ds(o_off(t), CHUNK * D)], out_sems[slot]
            )
            if t + NBUF < T:
                in_d[t + NBUF] = pltpu.async_copy(
                    x_hbm.at[pl.ds(x_off(t + NBUF), CHUNK * D)], ib,
                    in_sems[slot],
                )
            if b == B - 1 and j + 2 < n_chunks:
                pe_d[j + 2] = pe_load(j + 2)

        for t in range(max(0, T - NBUF), T):
            out_d[t].wait()

    return pl.kernel(
        body,
        out_type=jax.ShapeDtypeStruct((B * sc_rows * D,), jnp.float32),
        mesh=mesh,
        scratch_types=(
            [pltpu.VMEM((CHUNK * D,), jnp.float32)] * (2 * NBUF + 2)
            + [pltpu.SemaphoreType.DMA] * (2 * NBUF + 2)
        ),
    )


def kernel(x, pe_table):
    B, L, D = x.shape
    sc_out = _make_sc(B, L, D, L - 4096)(x.reshape(-1), pe_table.reshape(-1))
    return sc_out


# speedup vs baseline: 2.2050x; 2.2050x over previous
import jax
import jax.numpy as jnp
from jax import lax
from jax.experimental import pallas as pl
from jax.experimental.pallas import tpu as pltpu
from jax.experimental.pallas import tpu_sc as plsc

BL = 2048

NC, NS = 2, 16
NW = NC * NS
CHUNK = 8
SC_ROWS = 2048


def _add_kernel(x_ref, pe_ref, o_ref):
    o_ref[...] = x_ref[...] + pe_ref[...]


def _tc_partial(x, pe_table, L1):
    B, L, D = x.shape
    grid = (L1 // BL, B)
    return pl.pallas_call(
        _add_kernel,
        grid=grid,
        in_specs=[
            pl.BlockSpec((1, BL, D), lambda i, b: (b, i, 0)),
            pl.BlockSpec((BL, D), lambda i, b: (i, 0)),
        ],
        out_specs=pl.BlockSpec((1, BL, D), lambda i, b: (b, i, 0)),
        out_shape=jax.ShapeDtypeStruct((B, L, D), x.dtype),
    )(x, pe_table)


NBUF = 4


def _make_sc(B, L, D, L1):
    sc_rows = L - L1
    rows_per_w = sc_rows // NW
    n_chunks = rows_per_w // CHUNK
    T = n_chunks * B
    n_unroll = CHUNK * D // 16 // 8
    mesh = plsc.VectorSubcoreMesh(core_axis_name="c", subcore_axis_name="s")

    def body(x_hbm, pe_hbm, o_hbm, *scratch):
        in_bufs = scratch[0:NBUF]
        out_bufs = scratch[NBUF:2 * NBUF]
        pe_bufs = scratch[2 * NBUF:2 * NBUF + 2]
        in_sems = scratch[2 * NBUF + 2:3 * NBUF + 2]
        out_sems = scratch[3 * NBUF + 2:4 * NBUF + 2]
        pe_sems = scratch[4 * NBUF + 2:4 * NBUF + 4]

        w = lax.axis_index("s") * NC + lax.axis_index("c")
        l_base = L1 + w * rows_per_w

        def x_off(t):
            j, b = divmod(t, B)
            return (b * L + l_base + j * CHUNK) * D

        def o_off(t):
            j, b = divmod(t, B)
            return (b * sc_rows + (l_base - L1) + j * CHUNK) * D

        def pe_load(j):
            return pltpu.async_copy(
                pe_hbm.at[pl.ds((l_base + j * CHUNK) * D, CHUNK * D)],
                pe_bufs[j % 2],
                pe_sems[j % 2],
            )

        pe_d = {0: pe_load(0)}
        if n_chunks > 1:
            pe_d[1] = pe_load(1)
        in_d = {}
        for t in range(min(NBUF, T)):
            in_d[t] = pltpu.async_copy(
                x_hbm.at[pl.ds(x_off(t), CHUNK * D)], in_bufs[t % NBUF],
                in_sems[t % NBUF],
            )

        out_d = {}
        for t in range(T):
            j, b = divmod(t, B)
            slot = t % NBUF
            if b == 0:
                pe_d[j].wait()
            in_d[t].wait()
            if t >= NBUF:
                out_d[t - NBUF].wait()

            ib, ob, pb = in_bufs[slot], out_bufs[slot], pe_bufs[j % 2]

            def add_body(i, c, ib=ib, ob=ob, pb=pb):
                base = i * 128
                for k in range(8):
                    s = pl.ds(base + k * 16, 16)
                    ob[s] = ib[s] + pb[s]
                return c

            lax.fori_loop(0, n_unroll, add_body, 0)

            out_d[t] = pltpu.async_copy(
                ob, o_hbm.at[pl.ds(o_off(t), CHUNK * D)], out_sems[slot]
            )
            if t + NBUF < T:
                in_d[t + NBUF] = pltpu.async_copy(
                    x_hbm.at[pl.ds(x_off(t + NBUF), CHUNK * D)], ib,
                    in_sems[slot],
                )
            if b == B - 1 and j + 2 < n_chunks:
                pe_d[j + 2] = pe_load(j + 2)

        for t in range(max(0, T - NBUF), T):
            out_d[t].wait()

    return pl.kernel(
        body,
        out_type=jax.ShapeDtypeStruct((B * sc_rows * D,), jnp.float32),
        mesh=mesh,
        scratch_types=(
            [pltpu.VMEM((CHUNK * D,), jnp.float32)] * (2 * NBUF + 2)
            + [pltpu.SemaphoreType.DMA] * (2 * NBUF + 2)
        ),
    )


def kernel(x, pe_table):
    B, L, D = x.shape
    sc_out = _make_sc(B, L, D, L - 4096)(x.reshape(-1), pe_table.reshape(-1))
    tc_out = _tc_partial(x, pe_table, L)
    tc_out, _ = lax.optimization_barrier((tc_out, sc_out))
    return tc_out
